# Initial kernel scaffold; baseline (speedup 1.0000x reference)
import jax
import jax.numpy as jnp
from jax.experimental import pallas as pl


def kernel(x, y):
    raise NotImplementedError("write your pallas kernel here")


# speedup vs baseline: 1382.4396x; 1382.4396x over previous
import functools

import jax
import jax.numpy as jnp
from jax import lax
from jax.experimental import pallas as pl
from jax.experimental.pallas import tpu as pltpu
from jax.experimental.pallas import tpu_sc as plsc

LAMBDA_CC = 10.0
BINS = 64
EPS = 1e-06

NC, NS, L = 2, 16, 16
NW = NC * NS
B, C, H, W = 16, 3, 512, 512
HW = H * W
HALF = HW // NC
P = 16384
NCH = HALF // P
VECS = P // L
HL = BINS * L
SL = C * L


def _sc_body(x_hbm, y_hbm, xh_out, yh_out, xs_out, ys_out,
             b0r, b0g, b0b, b1r, b1g, b1b, histl_v, sums_v, sem_a, sem_b):
    bufs = ((b0r, b0g, b0b), (b1r, b1g, b1b))
    cid = lax.axis_index("c")
    sid = lax.axis_index("s")
    wid = cid * NS + sid
    base = cid * HALF
    zero16 = jnp.zeros((L,), jnp.float32)
    ones16 = jnp.ones((L,), jnp.float32)
    lane = lax.iota(jnp.int32, L)

    def start_copies(src, chunk):
        bset = chunk % 2
        sem = sem_a if bset == 0 else sem_b
        return [
            pltpu.async_copy(
                src.at[pl.ds((sid * C + ch) * HW + base + chunk * P, P)],
                bufs[bset][ch], sem)
            for ch in range(C)
        ]

    def accum_chunk(bset, sums):
        def vbody(v, carry):
            sr, sg, sb = carry
            off = v * L
            r = (bufs[bset][0][pl.ds(off, L)] + 1.0) * 0.5
            g = (bufs[bset][1][pl.ds(off, L)] + 1.0) * 0.5
            b = (bufs[bset][2][pl.ds(off, L)] + 1.0) * 0.5
            sr = sr + r
            sg = sg + g
            sb = sb + b
            lum = r * 0.299 + g * 0.587 + b * 0.114
            t = lum * 64.0
            ti = t.astype(jnp.int32)
            tf = ti.astype(jnp.float32)
            idx = ti - jnp.where(tf == t, 1, 0)
            idx = jnp.minimum(jnp.maximum(idx, 0), BINS - 1)
            plsc.addupdate_scatter(histl_v, [idx * L + lane], ones16)
            return (sr, sg, sb)

        return lax.fori_loop(0, VECS, vbody, sums)

    def process(src, h_out, s_out):
        for i in range(BINS):
            histl_v[pl.ds(i * L, L)] = zero16
        cps = start_copies(src, 0)
        sums = (zero16, zero16, zero16)
        for chunk in range(NCH):
            if chunk + 1 < NCH:
                nxt = start_copies(src, chunk + 1)
            for h in cps:
                h.wait()
            sums = accum_chunk(chunk % 2, sums)
            if chunk + 1 < NCH:
                cps = nxt
        sr, sg, sb = sums
        sums_v[pl.ds(0, L)] = sr
        sums_v[pl.ds(L, L)] = sg
        sums_v[pl.ds(2 * L, L)] = sb
        pltpu.sync_copy(histl_v, h_out.at[pl.ds(wid * HL, HL)])
        pltpu.sync_copy(sums_v, s_out.at[pl.ds(wid * SL, SL)])

    process(x_hbm, xh_out, xs_out)
    process(y_hbm, yh_out, ys_out)


_sc_hist = functools.partial(
    pl.kernel,
    out_type=[
        jax.ShapeDtypeStruct((NW * HL,), jnp.float32),
        jax.ShapeDtypeStruct((NW * HL,), jnp.float32),
        jax.ShapeDtypeStruct((NW * SL,), jnp.float32),
        jax.ShapeDtypeStruct((NW * SL,), jnp.float32),
    ],
    mesh=plsc.VectorSubcoreMesh(
        core_axis_name="c", subcore_axis_name="s", num_cores=NC, num_subcores=NS
    ),
    compiler_params=pltpu.CompilerParams(needs_layout_passes=False),
    scratch_types=[
        pltpu.VMEM((P,), jnp.float32),
        pltpu.VMEM((P,), jnp.float32),
        pltpu.VMEM((P,), jnp.float32),
        pltpu.VMEM((P,), jnp.float32),
        pltpu.VMEM((P,), jnp.float32),
        pltpu.VMEM((P,), jnp.float32),
        pltpu.VMEM((HL,), jnp.float32),
        pltpu.VMEM((SL,), jnp.float32),
        pltpu.SemaphoreType.DMA,
        pltpu.SemaphoreType.DMA,
    ],
)(_sc_body)


def _tail_body(xh_ref, yh_ref, xs_ref, ys_ref, out_ref):
    hw = jnp.float32(HW)
    xh = jnp.sum(xh_ref[...], axis=2)
    yh = jnp.sum(yh_ref[...], axis=2)
    xs = jnp.sum(xs_ref[...], axis=2) / hw
    ys = jnp.sum(ys_ref[...], axis=2) / hw

    xr, xg, xb = xs[:, 0:1], xs[:, 1:2], xs[:, 2:3]
    yr, yg, yb = ys[:, 0:1], ys[:, 1:2], ys[:, 2:3]
    grey_world = jnp.mean(jnp.abs(xr - xg) + jnp.abs(xg - xb) + jnp.abs(xb - xr))

    xden = xr + xg + xb + EPS
    yden = yr + yg + yb + EPS
    ratio = (jnp.mean(jnp.abs(xr / xden - yr / yden))
             + jnp.mean(jnp.abs(xg / xden - yg / yden))
             + jnp.mean(jnp.abs(xb / xden - yb / yden))) / 3.0

    xn = (xh + EPS) / (jnp.sum(xh, axis=1, keepdims=True) + EPS * BINS)
    yn = (yh + EPS) / (jnp.sum(yh, axis=1, keepdims=True) + EPS * BINS)
    kl = jnp.sum(yn * (jnp.log(yn) - jnp.log(xn))) / B

    out_ref[0, 0] = LAMBDA_CC * (grey_world + ratio + kl)


def kernel(x, y):
    x1 = x.reshape(-1)
    y1 = y.reshape(-1)
    xh, yh, xs, ys = _sc_hist(x1, y1)
    xh4 = xh.reshape(NC, NS, BINS, L).transpose(1, 2, 0, 3).reshape(NS, BINS, NC * L)
    yh4 = yh.reshape(NC, NS, BINS, L).transpose(1, 2, 0, 3).reshape(NS, BINS, NC * L)
    xs4 = xs.reshape(NC, NS, C, L).transpose(1, 2, 0, 3).reshape(NS, C, NC * L)
    ys4 = ys.reshape(NC, NS, C, L).transpose(1, 2, 0, 3).reshape(NS, C, NC * L)
    loss = pl.pallas_call(
        _tail_body,
        out_shape=jax.ShapeDtypeStruct((1, 1), jnp.float32),
        out_specs=pl.BlockSpec(memory_space=pltpu.SMEM),
    )(xh4, yh4, xs4, ys4)
    return loss[0, 0]


# speedup vs baseline: 1394.4128x; 1.0087x over previous
import functools

import jax
import jax.numpy as jnp
from jax import lax
from jax.experimental import pallas as pl
from jax.experimental.pallas import tpu as pltpu
from jax.experimental.pallas import tpu_sc as plsc

LAMBDA_CC = 10.0
BINS = 64
EPS = 1e-06

NC, NS, L = 2, 16, 16
NW = NC * NS
B, C, H, W = 16, 3, 512, 512
HW = H * W
HALF = HW // NC
P = 16384
NCH = HALF // P
VECS = P // L
HL = BINS * L
SL = C * L


def _sc_body(x_hbm, y_hbm, xh_out, yh_out, xs_out, ys_out,
             b0r, b0g, b0b, b1r, b1g, b1b, histl_v, sums_v, sem_a, sem_b):
    bufs = ((b0r, b0g, b0b), (b1r, b1g, b1b))
    cid = lax.axis_index("c")
    sid = lax.axis_index("s")
    wid = cid * NS + sid
    base = cid * HALF
    zero16 = jnp.zeros((L,), jnp.float32)
    ones16 = jnp.ones((L,), jnp.float32)
    lane = lax.iota(jnp.int32, L)

    def start_copies(src, chunk):
        bset = chunk % 2
        sem = sem_a if bset == 0 else sem_b
        return [
            pltpu.async_copy(
                src.at[pl.ds((sid * C + ch) * HW + base + chunk * P, P)],
                bufs[bset][ch], sem)
            for ch in range(C)
        ]

    UNROLL = 8

    def accum_chunk(bset, sums):
        def vbody(v, carry):
            acc = list(carry)
            off = v * (L * UNROLL)
            for u in range(UNROLL):
                o = off + u * L
                r = (bufs[bset][0][pl.ds(o, L)] + 1.0) * 0.5
                g = (bufs[bset][1][pl.ds(o, L)] + 1.0) * 0.5
                b = (bufs[bset][2][pl.ds(o, L)] + 1.0) * 0.5
                k = 3 * (u % 2)
                acc[k] = acc[k] + r
                acc[k + 1] = acc[k + 1] + g
                acc[k + 2] = acc[k + 2] + b
                lum = r * 0.299 + g * 0.587 + b * 0.114
                t = lum * 64.0
                ti = t.astype(jnp.int32)
                tf = ti.astype(jnp.float32)
                idx = ti - jnp.where(tf == t, 1, 0)
                idx = jnp.minimum(jnp.maximum(idx, 0), BINS - 1)
                plsc.addupdate_scatter(histl_v, [idx * L + lane], ones16)
            return tuple(acc)

        return lax.fori_loop(0, VECS // UNROLL, vbody, sums)

    def process(src, h_out, s_out):
        for i in range(BINS):
            histl_v[pl.ds(i * L, L)] = zero16
        cps = start_copies(src, 0)
        sums = (zero16,) * 6
        for chunk in range(NCH):
            if chunk + 1 < NCH:
                nxt = start_copies(src, chunk + 1)
            for h in cps:
                h.wait()
            sums = accum_chunk(chunk % 2, sums)
            if chunk + 1 < NCH:
                cps = nxt
        sums_v[pl.ds(0, L)] = sums[0] + sums[3]
        sums_v[pl.ds(L, L)] = sums[1] + sums[4]
        sums_v[pl.ds(2 * L, L)] = sums[2] + sums[5]
        pltpu.sync_copy(histl_v, h_out.at[pl.ds(wid * HL, HL)])
        pltpu.sync_copy(sums_v, s_out.at[pl.ds(wid * SL, SL)])

    process(x_hbm, xh_out, xs_out)
    process(y_hbm, yh_out, ys_out)


_sc_hist = functools.partial(
    pl.kernel,
    out_type=[
        jax.ShapeDtypeStruct((NW * HL,), jnp.float32),
        jax.ShapeDtypeStruct((NW * HL,), jnp.float32),
        jax.ShapeDtypeStruct((NW * SL,), jnp.float32),
        jax.ShapeDtypeStruct((NW * SL,), jnp.float32),
    ],
    mesh=plsc.VectorSubcoreMesh(
        core_axis_name="c", subcore_axis_name="s", num_cores=NC, num_subcores=NS
    ),
    compiler_params=pltpu.CompilerParams(needs_layout_passes=False),
    scratch_types=[
        pltpu.VMEM((P,), jnp.float32),
        pltpu.VMEM((P,), jnp.float32),
        pltpu.VMEM((P,), jnp.float32),
        pltpu.VMEM((P,), jnp.float32),
        pltpu.VMEM((P,), jnp.float32),
        pltpu.VMEM((P,), jnp.float32),
        pltpu.VMEM((HL,), jnp.float32),
        pltpu.VMEM((SL,), jnp.float32),
        pltpu.SemaphoreType.DMA,
        pltpu.SemaphoreType.DMA,
    ],
)(_sc_body)


def _tail_body(xh_ref, yh_ref, xs_ref, ys_ref, out_ref):
    hw = jnp.float32(HW)
    xh = jnp.sum(xh_ref[0], axis=2) + jnp.sum(xh_ref[1], axis=2)
    yh = jnp.sum(yh_ref[0], axis=2) + jnp.sum(yh_ref[1], axis=2)
    xs = (jnp.sum(xs_ref[0], axis=2) + jnp.sum(xs_ref[1], axis=2)) / hw
    ys = (jnp.sum(ys_ref[0], axis=2) + jnp.sum(ys_ref[1], axis=2)) / hw

    xr, xg, xb = xs[:, 0:1], xs[:, 1:2], xs[:, 2:3]
    yr, yg, yb = ys[:, 0:1], ys[:, 1:2], ys[:, 2:3]
    grey_world = jnp.mean(jnp.abs(xr - xg) + jnp.abs(xg - xb) + jnp.abs(xb - xr))

    xden = xr + xg + xb + EPS
    yden = yr + yg + yb + EPS
    ratio = (jnp.mean(jnp.abs(xr / xden - yr / yden))
             + jnp.mean(jnp.abs(xg / xden - yg / yden))
             + jnp.mean(jnp.abs(xb / xden - yb / yden))) / 3.0

    xn = (xh + EPS) / (jnp.sum(xh, axis=1, keepdims=True) + EPS * BINS)
    yn = (yh + EPS) / (jnp.sum(yh, axis=1, keepdims=True) + EPS * BINS)
    kl = jnp.sum(yn * (jnp.log(yn) - jnp.log(xn))) / B

    out_ref[0, 0] = LAMBDA_CC * (grey_world + ratio + kl)


def kernel(x, y):
    x1 = x.reshape(-1)
    y1 = y.reshape(-1)
    xh, yh, xs, ys = _sc_hist(x1, y1)
    xh4 = xh.reshape(NC, NS, BINS, L)
    yh4 = yh.reshape(NC, NS, BINS, L)
    xs4 = xs.reshape(NC, NS, C, L)
    ys4 = ys.reshape(NC, NS, C, L)
    loss = pl.pallas_call(
        _tail_body,
        out_shape=jax.ShapeDtypeStruct((1, 1), jnp.float32),
        out_specs=pl.BlockSpec(memory_space=pltpu.SMEM),
    )(xh4, yh4, xs4, ys4)
    return loss[0, 0]


# speedup vs baseline: 2962.1695x; 2.1243x over previous
import functools

import jax
import jax.numpy as jnp
from jax import lax
from jax.experimental import pallas as pl
from jax.experimental.pallas import tpu as pltpu
from jax.experimental.pallas import tpu_sc as plsc

LAMBDA_CC = 10.0
BINS = 64
EPS = 1e-06

NC, NS, L = 2, 16, 16
NW = NC * NS
B, C, H, W = 16, 3, 512, 512
HW = H * W
HALF = HW // NC
P = 16384
NCH = HALF // P
VECS = P // L
HL = BINS * L
SL = C * L


def _sc_body(x_hbm, y_hbm, xh_out, yh_out, xs_out, ys_out,
             b0r, b0g, b0b, b1r, b1g, b1b, histl_v, sums_v, sem_a, sem_b):
    bufs = ((b0r, b0g, b0b), (b1r, b1g, b1b))
    cid = lax.axis_index("c")
    sid = lax.axis_index("s")
    wid = cid * NS + sid
    base = cid * HALF
    zero16 = jnp.zeros((L,), jnp.float32)
    ones16 = jnp.ones((L,), jnp.float32)
    lane = lax.iota(jnp.int32, L)

    def start_copies(src, chunk):
        bset = chunk % 2
        sem = sem_a if bset == 0 else sem_b
        return [
            pltpu.async_copy(
                src.at[pl.ds((sid * C + ch) * HW + base + chunk * P, P)],
                bufs[bset][ch], sem)
            for ch in range(C)
        ]

    def accum_chunk(bset, sums):
        def vbody(v, carry):
            sr, sg, sb = carry
            off = v * L
            r = (bufs[bset][0][pl.ds(off, L)] + 1.0) * 0.5
            g = (bufs[bset][1][pl.ds(off, L)] + 1.0) * 0.5
            b = (bufs[bset][2][pl.ds(off, L)] + 1.0) * 0.5
            lum = r * 0.299 + g * 0.587 + b * 0.114
            t = lum * 64.0
            ti = t.astype(jnp.int32)
            tf = ti.astype(jnp.float32)
            idx = ti - jnp.where(tf == t, 1, 0)
            idx = jnp.minimum(jnp.maximum(idx, 0), BINS - 1)
            plsc.addupdate_scatter(histl_v, [idx * L + lane], ones16)
            return (sr + r, sg + g, sb + b)

        return plsc.parallel_loop(0, VECS, 1, unroll=8, carry=sums)(vbody)

    def process(src, h_out, s_out):
        for i in range(BINS):
            histl_v[pl.ds(i * L, L)] = zero16
        cps = start_copies(src, 0)
        sums = (zero16,) * 3
        for chunk in range(NCH):
            if chunk + 1 < NCH:
                nxt = start_copies(src, chunk + 1)
            for h in cps:
                h.wait()
            sums = accum_chunk(chunk % 2, sums)
            if chunk + 1 < NCH:
                cps = nxt
        sums_v[pl.ds(0, L)] = sums[0]
        sums_v[pl.ds(L, L)] = sums[1]
        sums_v[pl.ds(2 * L, L)] = sums[2]
        pltpu.sync_copy(histl_v, h_out.at[pl.ds(wid * HL, HL)])
        pltpu.sync_copy(sums_v, s_out.at[pl.ds(wid * SL, SL)])

    process(x_hbm, xh_out, xs_out)
    process(y_hbm, yh_out, ys_out)


_sc_hist = functools.partial(
    pl.kernel,
    out_type=[
        jax.ShapeDtypeStruct((NW * HL,), jnp.float32),
        jax.ShapeDtypeStruct((NW * HL,), jnp.float32),
        jax.ShapeDtypeStruct((NW * SL,), jnp.float32),
        jax.ShapeDtypeStruct((NW * SL,), jnp.float32),
    ],
    mesh=plsc.VectorSubcoreMesh(
        core_axis_name="c", subcore_axis_name="s", num_cores=NC, num_subcores=NS
    ),
    compiler_params=pltpu.CompilerParams(needs_layout_passes=False),
    scratch_types=[
        pltpu.VMEM((P,), jnp.float32),
        pltpu.VMEM((P,), jnp.float32),
        pltpu.VMEM((P,), jnp.float32),
        pltpu.VMEM((P,), jnp.float32),
        pltpu.VMEM((P,), jnp.float32),
        pltpu.VMEM((P,), jnp.float32),
        pltpu.VMEM((HL,), jnp.float32),
        pltpu.VMEM((SL,), jnp.float32),
        pltpu.SemaphoreType.DMA,
        pltpu.SemaphoreType.DMA,
    ],
)(_sc_body)


def _tail_body(xh_ref, yh_ref, xs_ref, ys_ref, out_ref):
    hw = jnp.float32(HW)
    xh = jnp.sum(xh_ref[0], axis=2) + jnp.sum(xh_ref[1], axis=2)
    yh = jnp.sum(yh_ref[0], axis=2) + jnp.sum(yh_ref[1], axis=2)
    xs = (jnp.sum(xs_ref[0], axis=2) + jnp.sum(xs_ref[1], axis=2)) / hw
    ys = (jnp.sum(ys_ref[0], axis=2) + jnp.sum(ys_ref[1], axis=2)) / hw

    xr, xg, xb = xs[:, 0:1], xs[:, 1:2], xs[:, 2:3]
    yr, yg, yb = ys[:, 0:1], ys[:, 1:2], ys[:, 2:3]
    grey_world = jnp.mean(jnp.abs(xr - xg) + jnp.abs(xg - xb) + jnp.abs(xb - xr))

    xden = xr + xg + xb + EPS
    yden = yr + yg + yb + EPS
    ratio = (jnp.mean(jnp.abs(xr / xden - yr / yden))
             + jnp.mean(jnp.abs(xg / xden - yg / yden))
             + jnp.mean(jnp.abs(xb / xden - yb / yden))) / 3.0

    xn = (xh + EPS) / (jnp.sum(xh, axis=1, keepdims=True) + EPS * BINS)
    yn = (yh + EPS) / (jnp.sum(yh, axis=1, keepdims=True) + EPS * BINS)
    kl = jnp.sum(yn * (jnp.log(yn) - jnp.log(xn))) / B

    out_ref[0, 0] = LAMBDA_CC * (grey_world + ratio + kl)


def kernel(x, y):
    x1 = x.reshape(-1)
    y1 = y.reshape(-1)
    xh, yh, xs, ys = _sc_hist(x1, y1)
    xh4 = xh.reshape(NC, NS, BINS, L)
    yh4 = yh.reshape(NC, NS, BINS, L)
    xs4 = xs.reshape(NC, NS, C, L)
    ys4 = ys.reshape(NC, NS, C, L)
    loss = pl.pallas_call(
        _tail_body,
        out_shape=jax.ShapeDtypeStruct((1, 1), jnp.float32),
        out_specs=pl.BlockSpec(memory_space=pltpu.SMEM),
    )(xh4, yh4, xs4, ys4)
    return loss[0, 0]


# speedup vs baseline: 4703.6897x; 1.5879x over previous
import functools

import jax
import jax.numpy as jnp
from jax import lax
from jax.experimental import pallas as pl
from jax.experimental.pallas import tpu as pltpu
from jax.experimental.pallas import tpu_sc as plsc

LAMBDA_CC = 10.0
BINS = 64
EPS = 1e-06

NC, NS, L = 2, 16, 16
NW = NC * NS
B, C, H, W = 16, 3, 512, 512
HW = H * W
HALF = HW // NC
PR = 32
P = PR * W
NCH = HALF // P
VECS = P // L
CW = W // L
HL = BINS * L
SL = C * L


def _sc_body(x_hbm, y_hbm, xh_out, yh_out, xs_out, ys_out,
             b0r, b0g, b0b, b1r, b1g, b1b, histl_v, sums_v, sem_a, sem_b):
    bufs = ((b0r, b0g, b0b), (b1r, b1g, b1b))
    cid = lax.axis_index("c")
    sid = lax.axis_index("s")
    wid = cid * NS + sid
    base = cid * HALF
    zero16 = jnp.zeros((L,), jnp.float32)
    ones16 = jnp.ones((L,), jnp.float32)
    lane = lax.iota(jnp.int32, L)

    row0 = cid * (H // NC)

    def start_copies(src, chunk):
        bset = chunk % 2
        sem = sem_a if bset == 0 else sem_b
        return [
            pltpu.async_copy(
                src.at[sid * C + ch, pl.ds(row0 + chunk * PR, PR), :],
                bufs[bset][ch], sem)
            for ch in range(C)
        ]

    def accum_chunk(bset, sums):
        def vbody(v, carry):
            sr, sg, sb = carry
            rw = lax.shift_right_logical(v, 5)
            col = lax.shift_left(lax.bitwise_and(v, CW - 1), 4)
            r = (bufs[bset][0][rw, pl.ds(col, L)] + 1.0) * 0.5
            g = (bufs[bset][1][rw, pl.ds(col, L)] + 1.0) * 0.5
            b = (bufs[bset][2][rw, pl.ds(col, L)] + 1.0) * 0.5
            lum = r * 0.299 + g * 0.587 + b * 0.114
            t = lum * 64.0
            ti = t.astype(jnp.int32)
            tf = ti.astype(jnp.float32)
            idx = ti - jnp.where(tf == t, 1, 0)
            idx = jnp.minimum(jnp.maximum(idx, 0), BINS - 1)
            plsc.addupdate_scatter(histl_v, [idx * L + lane], ones16)
            return (sr + r, sg + g, sb + b)

        return plsc.parallel_loop(0, VECS, 1, unroll=8, carry=sums)(vbody)

    def process(src, h_out, s_out):
        for i in range(BINS):
            histl_v[pl.ds(i * L, L)] = zero16
        cps = start_copies(src, 0)
        sums = (zero16,) * 3
        for chunk in range(NCH):
            if chunk + 1 < NCH:
                nxt = start_copies(src, chunk + 1)
            for h in cps:
                h.wait()
            sums = accum_chunk(chunk % 2, sums)
            if chunk + 1 < NCH:
                cps = nxt
        sums_v[pl.ds(0, L)] = sums[0]
        sums_v[pl.ds(L, L)] = sums[1]
        sums_v[pl.ds(2 * L, L)] = sums[2]
        pltpu.sync_copy(histl_v, h_out.at[pl.ds(wid * HL, HL)])
        pltpu.sync_copy(sums_v, s_out.at[pl.ds(wid * SL, SL)])

    process(x_hbm, xh_out, xs_out)
    process(y_hbm, yh_out, ys_out)


_sc_hist = functools.partial(
    pl.kernel,
    out_type=[
        jax.ShapeDtypeStruct((NW * HL,), jnp.float32),
        jax.ShapeDtypeStruct((NW * HL,), jnp.float32),
        jax.ShapeDtypeStruct((NW * SL,), jnp.float32),
        jax.ShapeDtypeStruct((NW * SL,), jnp.float32),
    ],
    mesh=plsc.VectorSubcoreMesh(
        core_axis_name="c", subcore_axis_name="s", num_cores=NC, num_subcores=NS
    ),
    compiler_params=pltpu.CompilerParams(needs_layout_passes=False),
    scratch_types=[
        pltpu.VMEM((PR, W), jnp.float32),
        pltpu.VMEM((PR, W), jnp.float32),
        pltpu.VMEM((PR, W), jnp.float32),
        pltpu.VMEM((PR, W), jnp.float32),
        pltpu.VMEM((PR, W), jnp.float32),
        pltpu.VMEM((PR, W), jnp.float32),
        pltpu.VMEM((HL,), jnp.float32),
        pltpu.VMEM((SL,), jnp.float32),
        pltpu.SemaphoreType.DMA,
        pltpu.SemaphoreType.DMA,
    ],
)(_sc_body)


def _tail_body(xh_ref, yh_ref, xs_ref, ys_ref, out_ref):
    hw = jnp.float32(HW)
    xh = jnp.sum(xh_ref[0], axis=2) + jnp.sum(xh_ref[1], axis=2)
    yh = jnp.sum(yh_ref[0], axis=2) + jnp.sum(yh_ref[1], axis=2)
    xs = (jnp.sum(xs_ref[0], axis=2) + jnp.sum(xs_ref[1], axis=2)) / hw
    ys = (jnp.sum(ys_ref[0], axis=2) + jnp.sum(ys_ref[1], axis=2)) / hw

    xr, xg, xb = xs[:, 0:1], xs[:, 1:2], xs[:, 2:3]
    yr, yg, yb = ys[:, 0:1], ys[:, 1:2], ys[:, 2:3]
    grey_world = jnp.mean(jnp.abs(xr - xg) + jnp.abs(xg - xb) + jnp.abs(xb - xr))

    xden = xr + xg + xb + EPS
    yden = yr + yg + yb + EPS
    ratio = (jnp.mean(jnp.abs(xr / xden - yr / yden))
             + jnp.mean(jnp.abs(xg / xden - yg / yden))
             + jnp.mean(jnp.abs(xb / xden - yb / yden))) / 3.0

    xn = (xh + EPS) / (jnp.sum(xh, axis=1, keepdims=True) + EPS * BINS)
    yn = (yh + EPS) / (jnp.sum(yh, axis=1, keepdims=True) + EPS * BINS)
    kl = jnp.sum(yn * (jnp.log(yn) - jnp.log(xn))) / B

    out_ref[0, 0] = LAMBDA_CC * (grey_world + ratio + kl)


def kernel(x, y):
    x1 = x.reshape(B * C, H, W)
    y1 = y.reshape(B * C, H, W)
    xh, yh, xs, ys = _sc_hist(x1, y1)
    xh4 = xh.reshape(NC, NS, BINS, L)
    yh4 = yh.reshape(NC, NS, BINS, L)
    xs4 = xs.reshape(NC, NS, C, L)
    ys4 = ys.reshape(NC, NS, C, L)
    loss = pl.pallas_call(
        _tail_body,
        out_shape=jax.ShapeDtypeStruct((1, 1), jnp.float32),
        out_specs=pl.BlockSpec(memory_space=pltpu.SMEM),
    )(xh4, yh4, xs4, ys4)
    return loss[0, 0]
